# baseline (speedup 1.0000x reference)
import functools

import jax
import jax.numpy as jnp
from jax import lax
from jax.experimental import pallas as pl
from jax.experimental.pallas import tpu as pltpu
from jax.experimental.pallas import tpu_sc as plsc

B, A, P, V, D = 4096, 200, 8, 92, 64
N = B * A
CH = 128
RT = 2048


def _sc_gather(table128, idx):
    info = plsc.get_sparse_core_info()
    nw = info.num_cores * info.num_subcores
    per_w = N // nw
    n_ch = per_w // CH
    mesh = plsc.VectorSubcoreMesh(core_axis_name="c", subcore_axis_name="s")

    @functools.partial(
        pl.kernel, mesh=mesh,
        out_type=jax.ShapeDtypeStruct((N, 128), jnp.float32),
        scratch_types=[
            pltpu.VMEM((CH,), jnp.int32),
            pltpu.VMEM((CH, 128), jnp.float32),
            pltpu.SemaphoreType.DMA,
        ],
    )
    def k(table_hbm, idx_hbm, out_hbm, idx_v, rows_v, sem):
        wid = lax.axis_index("s") * info.num_cores + lax.axis_index("c")
        base = wid * per_w

        def body(i, _):
            off = base + i * CH
            pltpu.sync_copy(idx_hbm.at[pl.ds(off, CH)], idx_v)
            pltpu.async_copy(table_hbm.at[idx_v], rows_v, sem).wait()
            pltpu.sync_copy(rows_v, out_hbm.at[pl.ds(off, CH)])
            return _

        lax.fori_loop(0, n_ch, body, None)

    return k(table128, idx)


def _tc_body(elem_ref, prop_ref, w1_ref, b1_ref, w2_ref, b2_ref, out_ref):
    p = prop_ref[...].astype(jnp.bfloat16)
    h = jnp.dot(p, w1_ref[...], preferred_element_type=jnp.float32)
    h = jnp.maximum(h + b1_ref[...], 0.0).astype(jnp.bfloat16)
    pe = jnp.dot(h, w2_ref[...], preferred_element_type=jnp.float32)
    out_ref[...] = elem_ref[:, :D] + pe + b2_ref[...]


def kernel(atomic_numbers, atomic_properties, table, W1, b1, W2, b2):
    idx = (atomic_numbers.astype(jnp.int32) - 1).reshape(N)
    table128 = jnp.zeros((V, 128), jnp.float32).at[:, :D].set(table)
    elem = _sc_gather(table128, idx)

    props = atomic_properties.reshape(N, P)
    out = pl.pallas_call(
        _tc_body,
        grid=(N // RT,),
        in_specs=[
            pl.BlockSpec((RT, 128), lambda i: (i, 0)),
            pl.BlockSpec((RT, P), lambda i: (i, 0)),
            pl.BlockSpec((P, 32), lambda i: (0, 0)),
            pl.BlockSpec((1, 32), lambda i: (0, 0)),
            pl.BlockSpec((32, D), lambda i: (0, 0)),
            pl.BlockSpec((1, D), lambda i: (0, 0)),
        ],
        out_specs=pl.BlockSpec((RT, D), lambda i: (i, 0)),
        out_shape=jax.ShapeDtypeStruct((N, D), jnp.float32),
        compiler_params=pltpu.CompilerParams(
            dimension_semantics=("arbitrary",),
        ),
    )(elem, props, W1.astype(jnp.bfloat16), b1.reshape(1, 32),
      W2.astype(jnp.bfloat16), b2.reshape(1, D))
    return out.reshape(B, A, D)


# speedup vs baseline: 11.7352x; 11.7352x over previous
import jax
import jax.numpy as jnp
from jax.experimental import pallas as pl
from jax.experimental.pallas import tpu as pltpu

B, A, P, V, D = 4096, 200, 8, 92, 64
AT = 8
BT = 2048
H = 32


def _body(an_ref, prop_ref, tab_ref, w1bd_ref, b1bd_ref, w2bd_ref,
          out_ref):
    idx = an_ref[...] - 1
    props = prop_ref[...].reshape(AT * P, BT).astype(jnp.bfloat16)
    h = jnp.dot(w1bd_ref[...], props, preferred_element_type=jnp.float32)
    h = jnp.maximum(h + b1bd_ref[...], 0.0).astype(jnp.bfloat16)
    tab = tab_ref[...]
    for g in range(AT // 4):
        prop4 = jnp.dot(w2bd_ref[...], h[g * 4 * H:(g + 1) * 4 * H, :],
                        preferred_element_type=jnp.float32)
        for j in range(4):
            a = g * 4 + j
            idx_b = jnp.broadcast_to(idx[a:a + 1, :], (D, BT))
            elem = jnp.take_along_axis(tab, idx_b, axis=1)
            out_ref[a, :, :] = elem + prop4[j * D:(j + 1) * D, :]


def kernel(atomic_numbers, atomic_properties, table, W1, b1, W2, b2):
    anT = atomic_numbers.astype(jnp.int32).T
    propsT = jnp.transpose(atomic_properties, (1, 2, 0))
    tabT = jnp.zeros((D, 128), jnp.float32).at[:, :V].set(table.T + b2[:, None])
    w1bd = jnp.zeros((AT * H, AT * P), jnp.bfloat16)
    w2bd = jnp.zeros((4 * D, 4 * H), jnp.bfloat16)
    w1t = W1.T.astype(jnp.bfloat16)
    w2t = W2.T.astype(jnp.bfloat16)
    for i in range(AT):
        w1bd = w1bd.at[i * H:(i + 1) * H, i * P:(i + 1) * P].set(w1t)
    for i in range(4):
        w2bd = w2bd.at[i * D:(i + 1) * D, i * H:(i + 1) * H].set(w2t)
    b1bd = jnp.tile(b1, AT).reshape(AT * H, 1)

    outT = pl.pallas_call(
        _body,
        grid=(A // AT, B // BT),
        in_specs=[
            pl.BlockSpec((AT, BT), lambda i, j: (i, j)),
            pl.BlockSpec((AT, P, BT), lambda i, j: (i, 0, j)),
            pl.BlockSpec((D, 128), lambda i, j: (0, 0)),
            pl.BlockSpec((AT * H, AT * P), lambda i, j: (0, 0)),
            pl.BlockSpec((AT * H, 1), lambda i, j: (0, 0)),
            pl.BlockSpec((4 * D, 4 * H), lambda i, j: (0, 0)),
        ],
        out_specs=pl.BlockSpec((AT, D, BT), lambda i, j: (i, 0, j)),
        out_shape=jax.ShapeDtypeStruct((A, D, B), jnp.float32),
        compiler_params=pltpu.CompilerParams(
            dimension_semantics=("arbitrary", "arbitrary"),
        ),
    )(anT, propsT, tabT, w1bd, b1bd, w2bd)
    return jnp.transpose(outT, (2, 0, 1))


# speedup vs baseline: 11.8119x; 1.0065x over previous
import jax
import jax.numpy as jnp
from jax.experimental import pallas as pl
from jax.experimental.pallas import tpu as pltpu

B, A, P, V, D = 4096, 200, 8, 92, 64
AT = 8
BT = 4096
H = 32


def _body(an_ref, prop_ref, tab_ref, w1bd_ref, b1bd_ref, w2bd_ref,
          out_ref):
    idx = an_ref[...] - 1
    props = prop_ref[...].reshape(AT * P, BT).astype(jnp.bfloat16)
    h = jnp.dot(w1bd_ref[...], props, preferred_element_type=jnp.float32)
    h = jnp.maximum(h + b1bd_ref[...], 0.0).astype(jnp.bfloat16)
    tab = tab_ref[...]
    for g in range(AT // 4):
        prop4 = jnp.dot(w2bd_ref[...], h[g * 4 * H:(g + 1) * 4 * H, :],
                        preferred_element_type=jnp.float32)
        for j in range(4):
            a = g * 4 + j
            idx_b = jnp.broadcast_to(idx[a:a + 1, :], (D, BT))
            elem = jnp.take_along_axis(tab, idx_b, axis=1)
            out_ref[a, :, :] = elem + prop4[j * D:(j + 1) * D, :]


def kernel(atomic_numbers, atomic_properties, table, W1, b1, W2, b2):
    anT = atomic_numbers.astype(jnp.int32).T
    propsT = jnp.transpose(atomic_properties, (1, 2, 0))
    tabT = jnp.zeros((D, 128), jnp.float32).at[:, :V].set(table.T + b2[:, None])
    w1bd = jnp.zeros((AT * H, AT * P), jnp.bfloat16)
    w2bd = jnp.zeros((4 * D, 4 * H), jnp.bfloat16)
    w1t = W1.T.astype(jnp.bfloat16)
    w2t = W2.T.astype(jnp.bfloat16)
    for i in range(AT):
        w1bd = w1bd.at[i * H:(i + 1) * H, i * P:(i + 1) * P].set(w1t)
    for i in range(4):
        w2bd = w2bd.at[i * D:(i + 1) * D, i * H:(i + 1) * H].set(w2t)
    b1bd = jnp.tile(b1, AT).reshape(AT * H, 1)

    outT = pl.pallas_call(
        _body,
        grid=(A // AT, B // BT),
        in_specs=[
            pl.BlockSpec((AT, BT), lambda i, j: (i, j)),
            pl.BlockSpec((AT, P, BT), lambda i, j: (i, 0, j)),
            pl.BlockSpec((D, 128), lambda i, j: (0, 0)),
            pl.BlockSpec((AT * H, AT * P), lambda i, j: (0, 0)),
            pl.BlockSpec((AT * H, 1), lambda i, j: (0, 0)),
            pl.BlockSpec((4 * D, 4 * H), lambda i, j: (0, 0)),
        ],
        out_specs=pl.BlockSpec((AT, D, BT), lambda i, j: (i, 0, j)),
        out_shape=jax.ShapeDtypeStruct((A, D, B), jnp.float32),
        compiler_params=pltpu.CompilerParams(
            dimension_semantics=("arbitrary", "arbitrary"),
        ),
    )(anT, propsT, tabT, w1bd, b1bd, w2bd)
    return jnp.transpose(outT, (2, 0, 1))


# speedup vs baseline: 11.8190x; 1.0006x over previous
import jax
import jax.numpy as jnp
from jax.experimental import pallas as pl
from jax.experimental.pallas import tpu as pltpu

B, A, P, V, D = 4096, 200, 8, 92, 64
AT = 8
BT = 4096
H = 32


def _body(an_ref, prop_ref, tab_ref, w1bd_ref, b1bd_ref, w2bd_ref,
          out_ref):
    idx = an_ref[...] - 1
    props = prop_ref[...].reshape(AT * P, BT).astype(jnp.bfloat16)
    h = jnp.dot(w1bd_ref[...], props, preferred_element_type=jnp.float32)
    h = jnp.maximum(h + b1bd_ref[...], 0.0).astype(jnp.bfloat16)
    tab = tab_ref[...]
    for g in range(AT // 4):
        prop4 = jnp.dot(w2bd_ref[...], h[g * 4 * H:(g + 1) * 4 * H, :],
                        preferred_element_type=jnp.float32)
        for j in range(4):
            a = g * 4 + j
            idx_b = jnp.broadcast_to(idx[a:a + 1, :], (D, BT))
            elem = jnp.take_along_axis(tab, idx_b, axis=1)
            out_ref[a, :, :] = elem + prop4[j * D:(j + 1) * D, :]


def kernel(atomic_numbers, atomic_properties, table, W1, b1, W2, b2):
    anT = atomic_numbers.astype(jnp.int32).T
    propsT = jnp.transpose(atomic_properties, (1, 2, 0))
    tabT = jnp.zeros((D, 128), jnp.float32).at[:, :V].set(table.T + b2[:, None])
    w1bd = jnp.zeros((AT * H, AT * P), jnp.bfloat16)
    w2bd = jnp.zeros((4 * D, 4 * H), jnp.bfloat16)
    w1t = W1.T.astype(jnp.bfloat16)
    w2t = W2.T.astype(jnp.bfloat16)
    for i in range(AT):
        w1bd = w1bd.at[i * H:(i + 1) * H, i * P:(i + 1) * P].set(w1t)
    for i in range(4):
        w2bd = w2bd.at[i * D:(i + 1) * D, i * H:(i + 1) * H].set(w2t)
    b1bd = jnp.tile(b1, AT).reshape(AT * H, 1)

    outT = pl.pallas_call(
        _body,
        grid=(A // AT, B // BT),
        in_specs=[
            pl.BlockSpec((AT, BT), lambda i, j: (i, j)),
            pl.BlockSpec((AT, P, BT), lambda i, j: (i, 0, j)),
            pl.BlockSpec((D, 128), lambda i, j: (0, 0)),
            pl.BlockSpec((AT * H, AT * P), lambda i, j: (0, 0)),
            pl.BlockSpec((AT * H, 1), lambda i, j: (0, 0)),
            pl.BlockSpec((4 * D, 4 * H), lambda i, j: (0, 0)),
        ],
        out_specs=pl.BlockSpec((AT, D, BT), lambda i, j: (i, 0, j)),
        out_shape=jax.ShapeDtypeStruct((A, D, B), jnp.float32),
        compiler_params=pltpu.CompilerParams(
            dimension_semantics=("parallel", "parallel"),
        ),
    )(anT, propsT, tabT, w1bd, b1bd, w2bd)
    return jnp.transpose(outT, (2, 0, 1))
